# baseline (speedup 1.0000x reference)
import functools

import jax
import jax.numpy as jnp
from jax import lax
from jax.experimental import pallas as pl
from jax.experimental.pallas import tpu as pltpu
from jax.experimental.pallas import tpu_sc as plsc

NC = 2
NS = 16
LANES = 16
NW = NC * NS
TAB_W = 128


def _detile_body(x_ref, o_ref):
    t = jnp.transpose(x_ref[...])
    o_ref[...] = jnp.concatenate(
        [t, jnp.zeros_like(t)], axis=1)


def _make_detile(V, D, bv):
    grid = (V + bv - 1) // bv

    return pl.pallas_call(
        _detile_body,
        grid=(grid,),
        in_specs=[pl.BlockSpec((D, bv), lambda j: (0, j))],
        out_specs=pl.BlockSpec((bv, TAB_W), lambda j: (j, 0)),
        out_shape=jax.ShapeDtypeStruct((V, TAB_W), jnp.float32),
        compiler_params=pltpu.CompilerParams(
            dimension_semantics=("parallel",)),
    )


def _make_sc_kernel(n_rows, D, chunk, n_chunks_per_w, sub, n_sub):
    mesh = plsc.VectorSubcoreMesh(core_axis_name="c", subcore_axis_name="s")

    @functools.partial(
        pl.kernel,
        mesh=mesh,
        compiler_params=pltpu.CompilerParams(use_tc_tiling_on_sc=False),
        out_type=jax.ShapeDtypeStruct((n_rows, TAB_W), jnp.float32),
        scratch_types=[
            pltpu.VMEM((chunk,), jnp.int32),
            pltpu.VMEM((chunk,), jnp.int32),
            pltpu.VMEM((chunk, TAB_W), jnp.float32),
            pltpu.VMEM((chunk, TAB_W), jnp.float32),
            pltpu.VMEM((chunk, D), jnp.float32),
            pltpu.SemaphoreType.DMA,
            pltpu.SemaphoreType.DMA,
            pltpu.SemaphoreType.DMA,
            pltpu.SemaphoreType.DMA,
        ],
    )
    def sc_kernel(xf_hbm, tab_hbm, pe_hbm, out_hbm,
                  idx0, idx1, rows0, rows1, pe_v,
                  gsem0, gsem1, wsem0, wsem1):
        wid = lax.axis_index("s") * NC + lax.axis_index("c")
        base_row = wid * (n_chunks_per_w * chunk)

        pltpu.sync_copy(pe_hbm, pe_v)

        idx_b = (idx0, idx1)
        rows_b = (rows0, rows1)
        gsem_b = (gsem0, gsem1)
        wsem_b = (wsem0, wsem1)

        def start_fetch(g, slot):
            start = base_row + g * chunk
            pltpu.sync_copy(xf_hbm.at[pl.ds(start, chunk)], idx_b[slot])
            cps = []
            for j in range(n_sub):
                cps.append(pltpu.async_copy(
                    tab_hbm.at[idx_b[slot].at[pl.ds(j * sub, sub)]],
                    rows_b[slot].at[pl.ds(j * sub, sub)],
                    gsem_b[slot]))
            return cps

        def add_pe(slot):
            rbuf = rows_b[slot]
            n_vec = D // LANES

            def body(r, carry):
                for c in range(n_vec):
                    sl = pl.ds(c * LANES, LANES)
                    rbuf[r, sl] = rbuf[r, sl] + pe_v[r, sl]
                return carry

            lax.fori_loop(0, chunk, body, 0)

        pending = [None, None]
        writes = [None, None]
        pending[0] = start_fetch(0, 0)
        for g in range(n_chunks_per_w):
            cur = g % 2
            nxt = (g + 1) % 2
            if g + 1 < n_chunks_per_w:
                if writes[nxt] is not None:
                    writes[nxt].wait()
                    writes[nxt] = None
                pending[nxt] = start_fetch(g + 1, nxt)
            for cp in pending[cur]:
                cp.wait()
            add_pe(cur)
            start = base_row + g * chunk
            writes[cur] = pltpu.async_copy(
                rows_b[cur], out_hbm.at[pl.ds(start, chunk)], wsem_b[cur])
        for w in writes:
            if w is not None:
                w.wait()

    return sc_kernel


def kernel(x, word_emb, pos_emb):
    B, S = x.shape
    V, D = word_emb.shape
    n_rows = B * S

    chunk = 2 * S
    n_chunks = n_rows // chunk
    n_chunks_per_w = n_chunks // NW
    assert chunk * n_chunks_per_w * NW == n_rows
    sub = 80
    n_sub = chunk // sub
    assert sub * n_sub == chunk

    xf = x.reshape(n_rows)
    pe = pos_emb[0, :S]
    pe2 = jnp.concatenate([pe, pe], axis=0)

    bv = 2048
    wt = _make_detile(V, D, bv)(word_emb.T)

    sc = _make_sc_kernel(n_rows, D, chunk, n_chunks_per_w, sub, n_sub)
    out_pad = sc(xf, wt, pe2)
    return out_pad.reshape(B, S, TAB_W)[:, :, :D]


# speedup vs baseline: 1.2213x; 1.2213x over previous
import functools

import jax
import jax.numpy as jnp
from jax import lax
from jax.experimental import pallas as pl
from jax.experimental.pallas import tpu as pltpu
from jax.experimental.pallas import tpu_sc as plsc

NC = 2
NS = 16
LANES = 16
NW = NC * NS
TAB_W = 128


def _detile_body(x_ref, o_ref):
    t = jnp.transpose(x_ref[...])
    o_ref[...] = jnp.concatenate(
        [t, jnp.zeros_like(t)], axis=1)


def _make_detile(V, D, bv):
    grid = (V + bv - 1) // bv

    return pl.pallas_call(
        _detile_body,
        grid=(grid,),
        in_specs=[pl.BlockSpec((D, bv), lambda j: (0, j))],
        out_specs=pl.BlockSpec((bv, TAB_W), lambda j: (j, 0)),
        out_shape=jax.ShapeDtypeStruct((V, TAB_W), jnp.float32),
        compiler_params=pltpu.CompilerParams(
            dimension_semantics=("parallel",)),
    )


def _make_sc_kernel(n_rows, D, chunk, n_chunks_per_w, sub, n_sub):
    mesh = plsc.VectorSubcoreMesh(core_axis_name="c", subcore_axis_name="s")

    @functools.partial(
        pl.kernel,
        mesh=mesh,
        compiler_params=pltpu.CompilerParams(use_tc_tiling_on_sc=False),
        out_type=jax.ShapeDtypeStruct((n_rows, TAB_W), jnp.float32),
        scratch_types=[
            pltpu.VMEM((chunk,), jnp.int32),
            pltpu.VMEM((chunk,), jnp.int32),
            pltpu.VMEM((chunk, TAB_W), jnp.float32),
            pltpu.VMEM((chunk, TAB_W), jnp.float32),
            pltpu.VMEM((chunk, D), jnp.float32),
            pltpu.SemaphoreType.DMA,
            pltpu.SemaphoreType.DMA,
            pltpu.SemaphoreType.DMA,
            pltpu.SemaphoreType.DMA,
        ],
    )
    def sc_kernel(xf_hbm, tab_hbm, pe_hbm, out_hbm,
                  idx0, idx1, rows0, rows1, pe_v,
                  gsem0, gsem1, wsem0, wsem1):
        wid = lax.axis_index("s") * NC + lax.axis_index("c")
        base_row = wid * (n_chunks_per_w * chunk)

        pltpu.sync_copy(pe_hbm, pe_v)

        idx_b = (idx0, idx1)
        rows_b = (rows0, rows1)
        gsem_b = (gsem0, gsem1)
        wsem_b = (wsem0, wsem1)

        def start_fetch(g, slot):
            start = base_row + g * chunk
            pltpu.sync_copy(xf_hbm.at[pl.ds(start, chunk)], idx_b[slot])
            cps = []
            for j in range(n_sub):
                cps.append(pltpu.async_copy(
                    tab_hbm.at[idx_b[slot].at[pl.ds(j * sub, sub)]],
                    rows_b[slot].at[pl.ds(j * sub, sub)],
                    gsem_b[slot]))
            return cps

        def add_pe(slot):
            rbuf = rows_b[slot]
            n_vec = D // LANES

            def body(r, carry):
                for c in range(n_vec):
                    sl = pl.ds(c * LANES, LANES)
                    rbuf[r, sl] = rbuf[r, sl] + pe_v[r, sl]
                return carry

            lax.fori_loop(0, chunk, body, 0)

        pending = [None, None]
        writes = [None, None]
        pending[0] = start_fetch(0, 0)
        for g in range(n_chunks_per_w):
            cur = g % 2
            nxt = (g + 1) % 2
            if g + 1 < n_chunks_per_w:
                if writes[nxt] is not None:
                    writes[nxt].wait()
                    writes[nxt] = None
                pending[nxt] = start_fetch(g + 1, nxt)
            for cp in pending[cur]:
                cp.wait()
            add_pe(cur)
            start = base_row + g * chunk
            writes[cur] = pltpu.async_copy(
                rows_b[cur], out_hbm.at[pl.ds(start, chunk)], wsem_b[cur])
        for w in writes:
            if w is not None:
                w.wait()

    return sc_kernel


def kernel(x, word_emb, pos_emb):
    B, S = x.shape
    V, D = word_emb.shape
    n_rows = B * S

    chunk = 2 * S
    n_chunks = n_rows // chunk
    n_chunks_per_w = n_chunks // NW
    assert chunk * n_chunks_per_w * NW == n_rows
    sub = 80
    n_sub = chunk // sub
    assert sub * n_sub == chunk

    xf = x.reshape(n_rows)
    pe = pos_emb[0, :S]
    pe2 = jnp.concatenate([pe, pe], axis=0)

    bv = 4096
    wt = _make_detile(V, D, bv)(word_emb.T)

    sc = _make_sc_kernel(n_rows, D, chunk, n_chunks_per_w, sub, n_sub)
    out_pad = sc(xf, wt, pe2)
    return out_pad.reshape(B, S, TAB_W)[:, :, :D]


# speedup vs baseline: 1.4016x; 1.1476x over previous
import functools

import jax
import jax.numpy as jnp
from jax import lax
from jax.experimental import pallas as pl
from jax.experimental.pallas import tpu as pltpu
from jax.experimental.pallas import tpu_sc as plsc

NC = 2
NS = 16
LANES = 16
NW = NC * NS
TAB_W = 128


def _detile_body(x_ref, o_ref):
    t = jnp.transpose(x_ref[...])
    o_ref[...] = jnp.concatenate(
        [t, jnp.zeros_like(t)], axis=1)


def _make_detile(V, D, bv):
    grid = (V + bv - 1) // bv

    return pl.pallas_call(
        _detile_body,
        grid=(grid,),
        in_specs=[pl.BlockSpec((D, bv), lambda j: (0, j))],
        out_specs=pl.BlockSpec((bv, TAB_W), lambda j: (j, 0)),
        out_shape=jax.ShapeDtypeStruct((V, TAB_W), jnp.float32),
        compiler_params=pltpu.CompilerParams(
            dimension_semantics=("parallel",)),
    )


def _make_sc_kernel(n_rows, D, chunk, n_chunks_per_w, sub, n_sub):
    mesh = plsc.VectorSubcoreMesh(core_axis_name="c", subcore_axis_name="s")

    @functools.partial(
        pl.kernel,
        mesh=mesh,
        compiler_params=pltpu.CompilerParams(use_tc_tiling_on_sc=False),
        out_type=jax.ShapeDtypeStruct((n_rows, TAB_W), jnp.float32),
        scratch_types=[
            pltpu.VMEM((chunk,), jnp.int32),
            pltpu.VMEM((chunk,), jnp.int32),
            pltpu.VMEM((chunk, TAB_W), jnp.float32),
            pltpu.VMEM((chunk, TAB_W), jnp.float32),
            pltpu.VMEM((chunk, D), jnp.float32),
            pltpu.SemaphoreType.DMA,
            pltpu.SemaphoreType.DMA,
            pltpu.SemaphoreType.DMA,
            pltpu.SemaphoreType.DMA,
        ],
    )
    def sc_kernel(xf_hbm, tab_hbm, pe_hbm, out_hbm,
                  idx0, idx1, rows0, rows1, pe_v,
                  gsem0, gsem1, wsem0, wsem1):
        wid = lax.axis_index("s") * NC + lax.axis_index("c")
        base_row = wid * (n_chunks_per_w * chunk)

        pltpu.sync_copy(pe_hbm, pe_v)

        idx_b = (idx0, idx1)
        rows_b = (rows0, rows1)
        gsem_b = (gsem0, gsem1)
        wsem_b = (wsem0, wsem1)

        def start_fetch(g, slot):
            start = base_row + g * chunk
            pltpu.sync_copy(xf_hbm.at[pl.ds(start, chunk)], idx_b[slot])
            cps = []
            for j in range(n_sub):
                cps.append(pltpu.async_copy(
                    tab_hbm.at[idx_b[slot].at[pl.ds(j * sub, sub)]],
                    rows_b[slot].at[pl.ds(j * sub, sub)],
                    gsem_b[slot]))
            return cps

        def add_pe(slot):
            rbuf = rows_b[slot]
            n_vec = D // LANES

            def body(r, carry):
                for c in range(n_vec):
                    sl = pl.ds(c * LANES, LANES)
                    rbuf[r, sl] = rbuf[r, sl] + pe_v[r, sl]
                return carry

            lax.fori_loop(0, chunk, body, 0)

        pending = [None, None]
        writes = [None, None]
        pending[0] = start_fetch(0, 0)
        for g in range(n_chunks_per_w):
            cur = g % 2
            nxt = (g + 1) % 2
            if g + 1 < n_chunks_per_w:
                if writes[nxt] is not None:
                    writes[nxt].wait()
                    writes[nxt] = None
                pending[nxt] = start_fetch(g + 1, nxt)
            for cp in pending[cur]:
                cp.wait()
            add_pe(cur)
            start = base_row + g * chunk
            writes[cur] = pltpu.async_copy(
                rows_b[cur], out_hbm.at[pl.ds(start, chunk)], wsem_b[cur])
        for w in writes:
            if w is not None:
                w.wait()

    return sc_kernel


def kernel(x, word_emb, pos_emb):
    B, S = x.shape
    V, D = word_emb.shape
    n_rows = B * S

    chunk = 2 * S
    n_chunks = n_rows // chunk
    n_chunks_per_w = n_chunks // NW
    assert chunk * n_chunks_per_w * NW == n_rows
    sub = 80
    n_sub = chunk // sub
    assert sub * n_sub == chunk

    xf = x.reshape(n_rows)
    pe = pos_emb[0, :S]
    pe2 = jnp.concatenate([pe, pe], axis=0)

    bv = 8192
    wt = _make_detile(V, D, bv)(word_emb.T)

    sc = _make_sc_kernel(n_rows, D, chunk, n_chunks_per_w, sub, n_sub)
    out_pad = sc(xf, wt, pe2)
    return out_pad.reshape(B, S, TAB_W)[:, :, :D]


# speedup vs baseline: 1.4587x; 1.0407x over previous
import functools

import jax
import jax.numpy as jnp
from jax import lax
from jax.experimental import pallas as pl
from jax.experimental.pallas import tpu as pltpu
from jax.experimental.pallas import tpu_sc as plsc

NC = 2
NS = 16
LANES = 16
NW = NC * NS
TAB_W = 128


def _detile_body(x_ref, o_ref):
    t = jnp.transpose(x_ref[...])
    o_ref[...] = jnp.concatenate(
        [t, jnp.zeros_like(t)], axis=1)


def _make_detile(V, D, bv):
    grid = (V + bv - 1) // bv

    return pl.pallas_call(
        _detile_body,
        grid=(grid,),
        in_specs=[pl.BlockSpec((D, bv), lambda j: (0, j))],
        out_specs=pl.BlockSpec((bv, TAB_W), lambda j: (j, 0)),
        out_shape=jax.ShapeDtypeStruct((V, TAB_W), jnp.float32),
        compiler_params=pltpu.CompilerParams(
            dimension_semantics=("parallel",)),
    )


def _make_sc_kernel(n_rows, D, chunk, n_chunks_per_w, sub, n_sub):
    mesh = plsc.VectorSubcoreMesh(core_axis_name="c", subcore_axis_name="s")

    @functools.partial(
        pl.kernel,
        mesh=mesh,
        compiler_params=pltpu.CompilerParams(use_tc_tiling_on_sc=False),
        out_type=jax.ShapeDtypeStruct((n_rows, TAB_W), jnp.float32),
        scratch_types=[
            pltpu.VMEM((chunk,), jnp.int32),
            pltpu.VMEM((chunk,), jnp.int32),
            pltpu.VMEM((chunk, TAB_W), jnp.float32),
            pltpu.VMEM((chunk, TAB_W), jnp.float32),
            pltpu.VMEM((chunk, D), jnp.float32),
            pltpu.SemaphoreType.DMA,
            pltpu.SemaphoreType.DMA,
            pltpu.SemaphoreType.DMA,
            pltpu.SemaphoreType.DMA,
        ],
    )
    def sc_kernel(xf_hbm, tab_hbm, pe_hbm, out_hbm,
                  idx0, idx1, rows0, rows1, pe_v,
                  gsem0, gsem1, wsem0, wsem1):
        wid = lax.axis_index("s") * NC + lax.axis_index("c")
        base_row = wid * (n_chunks_per_w * chunk)

        pltpu.sync_copy(pe_hbm, pe_v)

        idx_b = (idx0, idx1)
        rows_b = (rows0, rows1)
        gsem_b = (gsem0, gsem1)
        wsem_b = (wsem0, wsem1)

        def start_fetch(g, slot):
            start = base_row + g * chunk
            pltpu.sync_copy(xf_hbm.at[pl.ds(start, chunk)], idx_b[slot])
            cps = []
            for j in range(n_sub):
                cps.append(pltpu.async_copy(
                    tab_hbm.at[idx_b[slot].at[pl.ds(j * sub, sub)]],
                    rows_b[slot].at[pl.ds(j * sub, sub)],
                    gsem_b[slot]))
            return cps

        def add_pe(slot):
            rbuf = rows_b[slot]
            n_vec = D // LANES

            def body(r, carry):
                for c in range(n_vec):
                    sl = pl.ds(c * LANES, LANES)
                    rbuf[r, sl] = rbuf[r, sl] + pe_v[r, sl]
                return carry

            lax.fori_loop(0, chunk, body, 0)

        pending = [None, None]
        writes = [None, None]
        pending[0] = start_fetch(0, 0)
        for g in range(n_chunks_per_w):
            cur = g % 2
            nxt = (g + 1) % 2
            if g + 1 < n_chunks_per_w:
                if writes[nxt] is not None:
                    writes[nxt].wait()
                    writes[nxt] = None
                pending[nxt] = start_fetch(g + 1, nxt)
            for cp in pending[cur]:
                cp.wait()
            add_pe(cur)
            start = base_row + g * chunk
            writes[cur] = pltpu.async_copy(
                rows_b[cur], out_hbm.at[pl.ds(start, chunk)], wsem_b[cur])
        for w in writes:
            if w is not None:
                w.wait()

    return sc_kernel


def kernel(x, word_emb, pos_emb):
    B, S = x.shape
    V, D = word_emb.shape
    n_rows = B * S

    chunk = 2 * S
    n_chunks = n_rows // chunk
    n_chunks_per_w = n_chunks // NW
    assert chunk * n_chunks_per_w * NW == n_rows
    sub = 80
    n_sub = chunk // sub
    assert sub * n_sub == chunk

    xf = x.reshape(n_rows)
    pe = pos_emb[0, :S]
    pe2 = jnp.concatenate([pe, pe], axis=0)

    bv = 16384
    wt = _make_detile(V, D, bv)(word_emb.T)

    sc = _make_sc_kernel(n_rows, D, chunk, n_chunks_per_w, sub, n_sub)
    out_pad = sc(xf, wt, pe2)
    return out_pad.reshape(B, S, TAB_W)[:, :, :D]
